# baseline (speedup 1.0000x reference)
import functools

import jax
import jax.numpy as jnp
from jax import lax
from jax.experimental import pallas as pl
from jax.experimental.pallas import tpu as pltpu
from jax.experimental.pallas import tpu_sc as plsc


def _make_sc_gather(V, D, B):
    info = plsc.get_sparse_core_info()
    nw = info.num_cores * info.num_subcores
    assert D % info.num_lanes == 0 and B % (8 * nw) == 0
    b_per_w = B // nw
    mesh = plsc.VectorSubcoreMesh(core_axis_name="c", subcore_axis_name="s")

    @functools.partial(
        pl.kernel,
        mesh=mesh,
        out_type=jax.ShapeDtypeStruct((B, D), jnp.float32),
        compiler_params=pltpu.CompilerParams(use_tc_tiling_on_sc=False),
        scratch_types=[
            pltpu.VMEM((b_per_w,), jnp.int32),
            pltpu.VMEM((b_per_w, D), jnp.float32),
            pltpu.SemaphoreType.DMA,
        ],
    )
    def gather(table_hbm, idx_hbm, out_hbm, idx_v, rows_v, sem):
        wid = lax.axis_index("s") * info.num_cores + lax.axis_index("c")
        base = wid * b_per_w
        pltpu.sync_copy(idx_hbm.at[pl.ds(base, b_per_w)], idx_v)
        pltpu.async_copy(table_hbm.at[idx_v], rows_v, sem).wait()
        pltpu.sync_copy(rows_v, out_hbm.at[pl.ds(base, b_per_w)])

    return gather


def _dense_body(fe_ref, lw_ref, t2_ref, w1_ref, b1_ref, g1_ref, be1_ref,
                w2_ref, b2_ref, g2_ref, be2_ref, wfc_ref, bfc_ref, out_ref,
                *, lin_dim):
    f32 = jnp.float32
    fe = fe_ref[...]
    lz = jnp.dot(fe, lw_ref[...], preferred_element_type=f32)
    lp = jnp.dot(fe * fe, t2_ref[...], preferred_element_type=f32)
    w1 = w1_ref[...]
    y = (jnp.dot(lz, w1[:lin_dim], preferred_element_type=f32)
         + jnp.dot(lp, w1[lin_dim:], preferred_element_type=f32)
         + b1_ref[...])

    def bn_relu(y, g, b):
        m = jnp.mean(y, axis=0, keepdims=True)
        c = y - m
        v = jnp.mean(c * c, axis=0, keepdims=True)
        return jnp.maximum(g * c * lax.rsqrt(v + 1e-5) + b, 0.0)

    y = bn_relu(y, g1_ref[...], be1_ref[...])
    y = jnp.dot(y, w2_ref[...], preferred_element_type=f32) + b2_ref[...]
    y = bn_relu(y, g2_ref[...], be2_ref[...])
    out_ref[...] = jnp.dot(y, wfc_ref[...], preferred_element_type=f32) + bfc_ref[...]


def _dense(fe, lw, t2, w1, b1, g1, be1, w2, b2, g2, be2, wfc, bfc, lin_dim):
    B = fe.shape[0]
    return pl.pallas_call(
        functools.partial(_dense_body, lin_dim=lin_dim),
        out_shape=jax.ShapeDtypeStruct((B, 1), jnp.float32),
    )(fe, lw, t2, w1, b1, g1, be1, w2, b2, g2, be2, wfc, bfc)


def kernel(feat_index, feat_value, emb_table, linear_weights, theta,
           W1, b1, g1, be1, W2, b2, g2, be2, Wfc, bfc):
    B, F = feat_index.shape
    V, E = emb_table.shape
    lin_dim = linear_weights.shape[0]

    idx = feat_index.reshape(-1).astype(jnp.int32)
    fe = _make_sc_gather(V, E, B * F)(emb_table, idx)
    fe = fe.reshape(B, F * E)

    lw = linear_weights.reshape(lin_dim, F * E).T
    t2 = jnp.repeat(theta * theta, E, axis=1).T

    return _dense(fe, lw, t2,
                  W1, b1.reshape(1, -1), g1.reshape(1, -1), be1.reshape(1, -1),
                  W2, b2.reshape(1, -1), g2.reshape(1, -1), be2.reshape(1, -1),
                  Wfc, bfc.reshape(1, -1), lin_dim)
